# baseline (speedup 1.0000x reference)
import functools
import jax
import jax.numpy as jnp
from jax import lax
from jax.experimental import pallas as pl
from jax.experimental.pallas import tpu as pltpu
from jax.experimental.pallas import tpu_sc as plsc

_N = 1000000
_NC = 2
_NS = 16
_NW = _NC * _NS
_C = 8192
_NFULL = _N // _C
_TAIL = _N - _NFULL * _C
_KMAX = (_NFULL + _NW - 1) // _NW


def _sc_body(rad_hbm, mask_hbm, out_maxr_hbm, out_cnt_hbm,
             rad_v, mask_v, maxr_v, cnt_v):
    wid = lax.axis_index("s") * _NC + lax.axis_index("c")

    def group(g, carry):
        del carry
        w = mask_v[pl.ds(g * 16, 16)]
        base = g * 64
        for j in range(4):
            m = jnp.bitwise_and(lax.shift_right_logical(w, 8 * j), 1)
            mf = m.astype(jnp.float32)
            sub = base + 16 * j
            r = rad_v[pl.ds(sub, 16)]
            maxr_v[pl.ds(sub, 16)] = jnp.maximum(r, 0.0) * mf
            cnt_v[pl.ds(sub, 16)] = mf
        return 0

    for k in range(_KMAX):
        chunk = wid + k * _NW

        @pl.when(chunk < _NFULL)
        def _():
            off = chunk * _C
            pltpu.sync_copy(rad_hbm.at[pl.ds(off, _C)], rad_v)
            pltpu.sync_copy(mask_hbm.at[pl.ds(chunk * (_C // 4), _C // 4)],
                            mask_v)
            lax.fori_loop(0, _C // 64, group, 0)
            pltpu.sync_copy(maxr_v, out_maxr_hbm.at[pl.ds(off, _C)])
            pltpu.sync_copy(cnt_v, out_cnt_hbm.at[pl.ds(off, _C)])


_sc_update = functools.partial(
    pl.kernel,
    out_type=[jax.ShapeDtypeStruct((_N,), jnp.float32)] * 2,
    mesh=plsc.VectorSubcoreMesh(core_axis_name="c", subcore_axis_name="s",
                                num_cores=_NC, num_subcores=_NS),
    scratch_types=[
        pltpu.VMEM((_C,), jnp.float32),
        pltpu.VMEM((_C // 4,), jnp.int32),
        pltpu.VMEM((_C,), jnp.float32),
        pltpu.VMEM((_C,), jnp.float32),
    ],
)(_sc_body)


def _tail_block(maxr_in_ref, cnt_in_ref, rad_ref, m_ref,
                out_maxr_ref, out_cnt_ref):
    del maxr_in_ref, cnt_in_ref
    m = m_ref[...]
    rad = rad_ref[...]
    zero = jnp.zeros_like(rad)
    out_maxr_ref[...] = jnp.where(m, jnp.maximum(rad, zero), zero)
    out_cnt_ref[...] = m.astype(jnp.float32)


def _acc_block(g_ref, m_ref, out_acc_ref):
    gx = g_ref[0]
    gy = g_ref[1]
    gnorm = jnp.sqrt(gx * gx + gy * gy)
    out_acc_ref[...] = jnp.where(m_ref[...], gnorm, jnp.zeros_like(gnorm))


def kernel(max_radii2D, xyz_grad_accum, xyz_grad_count, radii,
           screenspace_gradient, visible_mask):
    n = max_radii2D.shape[0]
    sg_t = jnp.swapaxes(screenspace_gradient, 0, 1)

    mask_perm = (visible_mask.view(jnp.uint8)
                 .reshape(-1, 4, 16).swapaxes(1, 2)
                 .reshape(-1).view(jnp.int32))

    sc_maxr, sc_cnt = _sc_update(radii, mask_perm)

    tail_base = _NFULL * _C // 128
    tail_spec = pl.BlockSpec((128,), lambda i, b=tail_base: (b + i,))
    new_maxr, new_cnt = pl.pallas_call(
        _tail_block,
        grid=(5,),
        in_specs=[tail_spec] * 4,
        out_specs=[tail_spec] * 2,
        out_shape=[jax.ShapeDtypeStruct((n,), jnp.float32)] * 2,
        input_output_aliases={0: 0, 1: 1},
    )(sc_maxr, sc_cnt, radii, visible_mask)

    block = 131072
    grid = (n + block - 1) // block
    spec = pl.BlockSpec((block,), lambda i: (i,))
    g_spec = pl.BlockSpec((3, block), lambda i: (0, i))
    new_acc = pl.pallas_call(
        _acc_block,
        grid=(grid,),
        in_specs=[g_spec, spec],
        out_specs=spec,
        out_shape=jax.ShapeDtypeStruct((n,), jnp.float32),
    )(sg_t, visible_mask)

    return new_maxr, new_acc, new_cnt
